# baseline (speedup 1.0000x reference)
import jax
import jax.numpy as jnp
from jax.experimental import pallas as pl
from jax.experimental.pallas import tpu as pltpu

C = 128
H = 112
W = 152
INV_C = 1.0 / C
_NCH = 8
_CC = C // _NCH
_RR = _CC * 32


def _leaky(v):
    return jnp.where(v >= 0, v, 0.01 * v)


def _row_kernel(x_ref, xg_ref, wc_ref, bc_ref, w1_ref, w2_ref, wrc_ref,
                xo_ref, mask_ref, loss_ref):
    T = x_ref.shape[2]
    Bn = xo_ref.shape[1]
    X = x_ref[0]
    xg = xg_ref[0]
    idx = jnp.clip((xg * C).astype(jnp.int32), 0, C - 1)
    ci = jax.lax.broadcasted_iota(jnp.int32, (C, T), 0)
    oh = (ci == idx).astype(jnp.float32)

    cls = jnp.dot(wc_ref[0], X, preferred_element_type=jnp.float32)
    cls = _leaky(cls + bc_ref[0])
    mx = jnp.max(cls, axis=0, keepdims=True)
    e = jnp.exp(cls - mx)
    p = e / jnp.sum(e, axis=0, keepdims=True)
    lse = jnp.log(jnp.sum(jnp.exp(p), axis=0, keepdims=True))
    p_gt = jnp.sum(p * oh, axis=0, keepdims=True)
    loss_ref[0] = lse - p_gt

    y1 = jnp.zeros((32, T), jnp.float32)
    for k in range(_NCH):
        mk = jnp.dot(w1_ref[0, k * _RR:(k + 1) * _RR, :], X,
                     preferred_element_type=jnp.float32)
        mk3 = mk.reshape(_CC, 32, T)
        ohk = oh[k * _CC:(k + 1) * _CC, :]
        y1 = y1 + jnp.sum(mk3 * ohk[:, None, :], axis=0)
    y1 = _leaky(y1)

    g2 = jax.lax.dot_general(w2_ref[0], oh, (((0,), (0,)), ((), ())),
                             preferred_element_type=jnp.float32)
    g2v = g2.reshape(32, 32, T)
    y2 = _leaky(jnp.sum(g2v * y1[None, :, :], axis=1))

    g3 = jax.lax.dot_general(wrc_ref[0], oh, (((0,), (0,)), ((), ())),
                             preferred_element_type=jnp.float32)
    g3v = g3.reshape(2, 32, T)
    y3 = jnp.sum(g3v * y2[None, :, :], axis=1)
    reg = _leaky(y3[0])
    mask_ref[0] = _leaky(y3[1])[None, :]
    idxf = idx[0].astype(jnp.float32)

    for i in range(Bn):
        for j in range(Bn):
            xo_ref[0, i, j, :] = (idxf[i * W:(i + 1) * W] * INV_C
                                  + reg[j * W:(j + 1) * W] * INV_C)


_NSPLIT = 4
_HC = H // _NSPLIT


def _run_rows(x, x_gt, conv_c_w, conv_c_b, w1, w2, wrc, h0):
    B = x.shape[0]
    T = B * W
    Hc = _HC
    sl = slice(h0, h0 + Hc)
    xr = jnp.transpose(x[:, :, sl], (2, 1, 0, 3)).reshape(Hc, 128, T)
    xgr = jnp.transpose(x_gt[:, :, sl], (2, 1, 0, 3)).reshape(Hc, 1, T)
    wc = conv_c_w.reshape(H, C, 128)[sl]
    bc = conv_c_b.reshape(H, C)[sl, :, None]
    w1n = (w1.reshape(H, C, 128, 32)[sl]
           .transpose(0, 1, 3, 2).reshape(Hc, C * 32, 128))
    w2g = (w2.reshape(H, C, 32, 32)[sl]
           .transpose(0, 1, 3, 2).reshape(Hc, C, 32 * 32))
    wrcg = (wrc.reshape(H, C, 32, 2)[sl]
            .transpose(0, 1, 3, 2).reshape(Hc, C, 2 * 32))

    return pl.pallas_call(
        _row_kernel,
        grid=(Hc,),
        in_specs=[
            pl.BlockSpec((1, 128, T), lambda h: (h, 0, 0)),
            pl.BlockSpec((1, 1, T), lambda h: (h, 0, 0)),
            pl.BlockSpec((1, C, 128), lambda h: (h, 0, 0)),
            pl.BlockSpec((1, C, 1), lambda h: (h, 0, 0)),
            pl.BlockSpec((1, C * 32, 128), lambda h: (h, 0, 0)),
            pl.BlockSpec((1, C, 32 * 32), lambda h: (h, 0, 0)),
            pl.BlockSpec((1, C, 2 * 32), lambda h: (h, 0, 0)),
        ],
        out_specs=[
            pl.BlockSpec((1, B, B, W), lambda h: (h, 0, 0, 0)),
            pl.BlockSpec((1, 1, T), lambda h: (h, 0, 0)),
            pl.BlockSpec((1, 1, T), lambda h: (h, 0, 0)),
        ],
        out_shape=[
            jax.ShapeDtypeStruct((_HC, B, B, W), jnp.float32),
            jax.ShapeDtypeStruct((_HC, 1, T), jnp.float32),
            jax.ShapeDtypeStruct((_HC, 1, T), jnp.float32),
        ],
        compiler_params=pltpu.CompilerParams(
            dimension_semantics=("parallel",)),
    )(xr, xgr, wc, bc, w1n, w2g, wrcg)


def kernel(x, x_gt, conv_c_w, conv_c_b, w1, w2, wrc):
    B = x.shape[0]
    parts = [_run_rows(x, x_gt, conv_c_w, conv_c_b, w1, w2, wrc, k * _HC)
             for k in range(_NSPLIT)]
    xo_t = jnp.concatenate([p[0] for p in parts], axis=0)
    mask_t = jnp.concatenate([p[1] for p in parts], axis=0)
    loss_t = jnp.concatenate([p[2] for p in parts], axis=0)

    x_out = jnp.transpose(xo_t, (1, 2, 0, 3))
    mask = jnp.transpose(mask_t.reshape(H, B, W), (1, 0, 2))
    loss = jnp.transpose(loss_t.reshape(H, B, W), (1, 0, 2))
    return (x_out, mask, loss)


# speedup vs baseline: 1.4550x; 1.4550x over previous
import jax
import jax.numpy as jnp
from jax.experimental import pallas as pl
from jax.experimental.pallas import tpu as pltpu

C = 128
H = 112
W = 152
INV_C = 1.0 / C
_NCH = 8
_CC = C // _NCH
_RR = _CC * 32


def _leaky(v):
    return jnp.where(v >= 0, v, 0.01 * v)


def _row_kernel(x_ref, xg_ref, wc_ref, bc_ref, w1_ref, w2_ref, wrc_ref,
                xo_ref, mask_ref, loss_ref):
    T = x_ref.shape[2]
    Bn = xo_ref.shape[1]
    X = x_ref[0]
    xg = xg_ref[0]
    idx = jnp.clip((xg * C).astype(jnp.int32), 0, C - 1)
    ci = jax.lax.broadcasted_iota(jnp.int32, (C, T), 0)
    oh = (ci == idx).astype(jnp.float32)
    ohb = oh.astype(jnp.bfloat16)
    Xb = x_ref[0].astype(jnp.bfloat16)

    cls = jnp.dot(wc_ref[0], X, preferred_element_type=jnp.float32)
    cls = _leaky(cls + bc_ref[0])
    mx = jnp.max(cls, axis=0, keepdims=True)
    e = jnp.exp(cls - mx)
    p = e / jnp.sum(e, axis=0, keepdims=True)
    lse = jnp.log(jnp.sum(jnp.exp(p), axis=0, keepdims=True))
    p_gt = jnp.sum(p * oh, axis=0, keepdims=True)
    loss_ref[0] = lse - p_gt

    y1 = jnp.zeros((32, T), jnp.float32)
    for k in range(_NCH):
        mk = jnp.dot(w1_ref[0, k * _RR:(k + 1) * _RR, :], Xb,
                     preferred_element_type=jnp.float32)
        mk3 = mk.reshape(_CC, 32, T)
        ohk = oh[k * _CC:(k + 1) * _CC, :]
        y1 = y1 + jnp.sum(mk3 * ohk[:, None, :], axis=0)
    y1 = _leaky(y1)

    g2 = jax.lax.dot_general(w2_ref[0], ohb, (((0,), (0,)), ((), ())),
                             preferred_element_type=jnp.float32)
    g2v = g2.reshape(32, 32, T)
    y2 = _leaky(jnp.sum(g2v * y1[None, :, :], axis=1))

    g3 = jax.lax.dot_general(wrc_ref[0], ohb, (((0,), (0,)), ((), ())),
                             preferred_element_type=jnp.float32)
    g3v = g3.reshape(2, 32, T)
    y3 = jnp.sum(g3v * y2[None, :, :], axis=1)
    reg = _leaky(y3[0])
    mask_ref[0] = _leaky(y3[1])[None, :]
    idxf = idx[0].astype(jnp.float32)

    for i in range(Bn):
        for j in range(Bn):
            xo_ref[0, i, j, :] = (idxf[i * W:(i + 1) * W] * INV_C
                                  + reg[j * W:(j + 1) * W] * INV_C)


def kernel(x, x_gt, conv_c_w, conv_c_b, w1, w2, wrc):
    B = x.shape[0]
    T = B * W
    xr = jnp.transpose(x, (2, 1, 0, 3)).reshape(H, 128, T)
    xgr = jnp.transpose(x_gt, (2, 1, 0, 3)).reshape(H, 1, T)
    wc = conv_c_w.reshape(H, C, 128)
    bc = conv_c_b.reshape(H, C)[:, :, None]
    bf = jnp.bfloat16
    w1n = (w1.reshape(H, C, 128, 32).transpose(0, 1, 3, 2)
           .reshape(H, C * 32, 128).astype(bf))
    w2g = (w2.reshape(H, C, 32, 32).transpose(0, 1, 3, 2)
           .reshape(H, C, 32 * 32).astype(bf))
    wrcg = (wrc.reshape(H, C, 32, 2).transpose(0, 1, 3, 2)
            .reshape(H, C, 2 * 32).astype(bf))

    xo_t, mask_t, loss_t = pl.pallas_call(
        _row_kernel,
        grid=(H,),
        in_specs=[
            pl.BlockSpec((1, 128, T), lambda h: (h, 0, 0)),
            pl.BlockSpec((1, 1, T), lambda h: (h, 0, 0)),
            pl.BlockSpec((1, C, 128), lambda h: (h, 0, 0)),
            pl.BlockSpec((1, C, 1), lambda h: (h, 0, 0)),
            pl.BlockSpec((1, C * 32, 128), lambda h: (h, 0, 0)),
            pl.BlockSpec((1, C, 32 * 32), lambda h: (h, 0, 0)),
            pl.BlockSpec((1, C, 2 * 32), lambda h: (h, 0, 0)),
        ],
        out_specs=[
            pl.BlockSpec((1, B, B, W), lambda h: (h, 0, 0, 0)),
            pl.BlockSpec((1, 1, T), lambda h: (h, 0, 0)),
            pl.BlockSpec((1, 1, T), lambda h: (h, 0, 0)),
        ],
        out_shape=[
            jax.ShapeDtypeStruct((H, B, B, W), jnp.float32),
            jax.ShapeDtypeStruct((H, 1, T), jnp.float32),
            jax.ShapeDtypeStruct((H, 1, T), jnp.float32),
        ],
        compiler_params=pltpu.CompilerParams(
            dimension_semantics=("parallel",)),
    )(xr, xgr, wc, bc, w1n, w2g, wrcg)

    x_out = jnp.transpose(xo_t, (1, 2, 0, 3))
    mask = jnp.transpose(mask_t.reshape(H, B, W), (1, 0, 2))
    loss = jnp.transpose(loss_t.reshape(H, B, W), (1, 0, 2))
    return (x_out, mask, loss)


# speedup vs baseline: 1.6728x; 1.1497x over previous
import jax
import jax.numpy as jnp
from jax.experimental import pallas as pl
from jax.experimental.pallas import tpu as pltpu

C = 128
H = 112
W = 152
INV_C = 1.0 / C
_NCH = 8
_CC = C // _NCH
_RR = _CC * 32


def _leaky(v):
    return jnp.where(v >= 0, v, 0.01 * v)


def _row_kernel(x_ref, xg_ref, wc_ref, bc_ref, w1_ref, w2_ref, wrc_ref,
                xo_ref, mask_ref, loss_ref):
    T = x_ref.shape[2]
    Bn = xo_ref.shape[1]
    X = x_ref[0]
    xg = xg_ref[0]
    idx = jnp.clip((xg * C).astype(jnp.int32), 0, C - 1)
    ci = jax.lax.broadcasted_iota(jnp.int32, (C, T), 0)
    oh = (ci == idx).astype(jnp.float32)

    cls = jnp.dot(wc_ref[0], X, preferred_element_type=jnp.float32)
    cls = _leaky(cls + bc_ref[0])
    mx = jnp.max(cls, axis=0, keepdims=True)
    e = jnp.exp(cls - mx)
    p = e / jnp.sum(e, axis=0, keepdims=True)
    lse = jnp.log(jnp.sum(jnp.exp(p), axis=0, keepdims=True))
    p_gt = jnp.sum(p * oh, axis=0, keepdims=True)
    loss_ref[0] = lse - p_gt

    y1 = jnp.zeros((32, T), jnp.float32)
    for k in range(_NCH):
        mk = jnp.dot(w1_ref[0, k * _RR:(k + 1) * _RR, :], X,
                     preferred_element_type=jnp.float32)
        mk3 = mk.reshape(_CC, 32, T)
        ohk = oh[k * _CC:(k + 1) * _CC, :]
        y1 = y1 + jnp.sum(mk3 * ohk[:, None, :], axis=0)
    y1 = _leaky(y1)

    g2 = jax.lax.dot_general(w2_ref[0], oh, (((0,), (0,)), ((), ())),
                             preferred_element_type=jnp.float32)
    g2v = g2.reshape(32, 32, T)
    y2 = _leaky(jnp.sum(g2v * y1[None, :, :], axis=1))

    g3 = jax.lax.dot_general(wrc_ref[0], oh, (((0,), (0,)), ((), ())),
                             preferred_element_type=jnp.float32)
    g3v = g3.reshape(2, 32, T)
    y3 = jnp.sum(g3v * y2[None, :, :], axis=1)
    reg = _leaky(y3[0])
    mask_ref[0] = _leaky(y3[1])[None, :]
    idxf = idx[0].astype(jnp.float32)

    for i in range(Bn):
        for j in range(Bn):
            xo_ref[0, i, j, :] = (idxf[i * W:(i + 1) * W] * INV_C
                                  + reg[j * W:(j + 1) * W] * INV_C)


def kernel(x, x_gt, conv_c_w, conv_c_b, w1, w2, wrc):
    B = x.shape[0]
    T = B * W
    xr = jnp.transpose(x, (2, 1, 0, 3)).reshape(H, 128, T)
    xgr = jnp.transpose(x_gt, (2, 1, 0, 3)).reshape(H, 1, T)
    wc = conv_c_w.reshape(H, C, 128)
    bc = conv_c_b.reshape(H, C)[:, :, None]
    w1n = w1.reshape(H, C, 128, 32).transpose(0, 1, 3, 2).reshape(H, C * 32, 128)
    w2g = w2.reshape(H, C, 32, 32).transpose(0, 1, 3, 2).reshape(H, C, 32 * 32)
    wrcg = wrc.reshape(H, C, 32, 2).transpose(0, 1, 3, 2).reshape(H, C, 2 * 32)

    xo_t, mask_t, loss_t = pl.pallas_call(
        _row_kernel,
        grid=(H,),
        in_specs=[
            pl.BlockSpec((1, 128, T), lambda h: (h, 0, 0)),
            pl.BlockSpec((1, 1, T), lambda h: (h, 0, 0)),
            pl.BlockSpec((1, C, 128), lambda h: (h, 0, 0)),
            pl.BlockSpec((1, C, 1), lambda h: (h, 0, 0)),
            pl.BlockSpec((1, C * 32, 128), lambda h: (h, 0, 0)),
            pl.BlockSpec((1, C, 32 * 32), lambda h: (h, 0, 0)),
            pl.BlockSpec((1, C, 2 * 32), lambda h: (h, 0, 0)),
        ],
        out_specs=[
            pl.BlockSpec((1, B, B, W), lambda h: (h, 0, 0, 0)),
            pl.BlockSpec((1, 1, T), lambda h: (h, 0, 0)),
            pl.BlockSpec((1, 1, T), lambda h: (h, 0, 0)),
        ],
        out_shape=[
            jax.ShapeDtypeStruct((H, B, B, W), jnp.float32),
            jax.ShapeDtypeStruct((H, 1, T), jnp.float32),
            jax.ShapeDtypeStruct((H, 1, T), jnp.float32),
        ],
        compiler_params=pltpu.CompilerParams(
            dimension_semantics=("parallel",)),
    )(xr, xgr, wc, bc, w1n, w2g, wrcg)

    x_out = jnp.transpose(xo_t, (1, 2, 0, 3))
    mask = jnp.transpose(mask_t.reshape(H, B, W), (1, 0, 2))
    loss = jnp.transpose(loss_t.reshape(H, B, W), (1, 0, 2))
    return (x_out, mask, loss)


# speedup vs baseline: 1.8418x; 1.1010x over previous
import jax
import jax.numpy as jnp
from jax.experimental import pallas as pl
from jax.experimental.pallas import tpu as pltpu

C = 128
H = 112
W = 152
INV_C = 1.0 / C
_NCH = 8
_CC = C // _NCH
_RR = _CC * 32


def _leaky(v):
    return jnp.where(v >= 0, v, 0.01 * v)


def _row_kernel(x_ref, xg_ref, wc_ref, bc_ref, w1_ref, w2_ref, wrc_ref,
                xo_ref, mask_ref, loss_ref):
    T = x_ref.shape[2]
    Bn = xo_ref.shape[1]
    for r in range(x_ref.shape[0]):
        _one_row(r, x_ref, xg_ref, wc_ref, bc_ref, w1_ref, w2_ref, wrc_ref,
                 xo_ref, mask_ref, loss_ref, T, Bn)


def _one_row(r, x_ref, xg_ref, wc_ref, bc_ref, w1_ref, w2_ref, wrc_ref,
             xo_ref, mask_ref, loss_ref, T, Bn):
    X = x_ref[r]
    xg = xg_ref[r]
    idx = jnp.clip((xg * C).astype(jnp.int32), 0, C - 1)
    ci = jax.lax.broadcasted_iota(jnp.int32, (C, T), 0)
    oh = (ci == idx).astype(jnp.float32)

    cls = jnp.dot(wc_ref[r], X, preferred_element_type=jnp.float32)
    cls = _leaky(cls + bc_ref[r])
    mx = jnp.max(cls, axis=0, keepdims=True)
    e = jnp.exp(cls - mx)
    p = e / jnp.sum(e, axis=0, keepdims=True)
    lse = jnp.log(jnp.sum(jnp.exp(p), axis=0, keepdims=True))
    p_gt = jnp.sum(p * oh, axis=0, keepdims=True)
    loss_ref[r] = lse - p_gt

    y1 = jnp.zeros((32, T), jnp.float32)
    for k in range(_NCH):
        mk = jnp.dot(w1_ref[r, k * _RR:(k + 1) * _RR, :], X,
                     preferred_element_type=jnp.float32)
        mk3 = mk.reshape(_CC, 32, T)
        ohk = oh[k * _CC:(k + 1) * _CC, :]
        y1 = y1 + jnp.sum(mk3 * ohk[:, None, :], axis=0)
    y1 = _leaky(y1)

    g2 = jax.lax.dot_general(w2_ref[r], oh, (((0,), (0,)), ((), ())),
                             preferred_element_type=jnp.float32)
    g2v = g2.reshape(32, 32, T)
    y2 = _leaky(jnp.sum(g2v * y1[None, :, :], axis=1))

    g3 = jax.lax.dot_general(wrc_ref[r], oh, (((0,), (0,)), ((), ())),
                             preferred_element_type=jnp.float32)
    g3v = g3.reshape(2, 32, T)
    y3 = jnp.sum(g3v * y2[None, :, :], axis=1)
    reg = _leaky(y3[0])
    mask_ref[r] = _leaky(y3[1])[None, :]
    idxf = idx[0].astype(jnp.float32)

    for i in range(Bn):
        for j in range(Bn):
            xo_ref[r, i, j, :] = (idxf[i * W:(i + 1) * W] * INV_C
                                  + reg[j * W:(j + 1) * W] * INV_C)


def kernel(x, x_gt, conv_c_w, conv_c_b, w1, w2, wrc):
    B = x.shape[0]
    T = B * W
    xr = jnp.transpose(x, (2, 1, 0, 3)).reshape(H, 128, T)
    xgr = jnp.transpose(x_gt, (2, 1, 0, 3)).reshape(H, 1, T)
    wc = conv_c_w.reshape(H, C, 128)
    bc = conv_c_b.reshape(H, C)[:, :, None]
    w1n = w1.reshape(H, C, 128, 32).transpose(0, 1, 3, 2).reshape(H, C * 32, 128)
    w2g = w2.reshape(H, C, 32, 32).transpose(0, 1, 3, 2).reshape(H, C, 32 * 32)
    wrcg = wrc.reshape(H, C, 32, 2).transpose(0, 1, 3, 2).reshape(H, C, 2 * 32)

    R = 2
    xo_t, mask_t, loss_t = pl.pallas_call(
        _row_kernel,
        grid=(H // R,),
        in_specs=[
            pl.BlockSpec((R, 128, T), lambda h: (h, 0, 0)),
            pl.BlockSpec((R, 1, T), lambda h: (h, 0, 0)),
            pl.BlockSpec((R, C, 128), lambda h: (h, 0, 0)),
            pl.BlockSpec((R, C, 1), lambda h: (h, 0, 0)),
            pl.BlockSpec((R, C * 32, 128), lambda h: (h, 0, 0)),
            pl.BlockSpec((R, C, 32 * 32), lambda h: (h, 0, 0)),
            pl.BlockSpec((R, C, 2 * 32), lambda h: (h, 0, 0)),
        ],
        out_specs=[
            pl.BlockSpec((R, B, B, W), lambda h: (h, 0, 0, 0)),
            pl.BlockSpec((R, 1, T), lambda h: (h, 0, 0)),
            pl.BlockSpec((R, 1, T), lambda h: (h, 0, 0)),
        ],
        out_shape=[
            jax.ShapeDtypeStruct((H, B, B, W), jnp.float32),
            jax.ShapeDtypeStruct((H, 1, T), jnp.float32),
            jax.ShapeDtypeStruct((H, 1, T), jnp.float32),
        ],
        compiler_params=pltpu.CompilerParams(
            dimension_semantics=("parallel",)),
    )(xr, xgr, wc, bc, w1n, w2g, wrcg)

    x_out = jnp.transpose(xo_t, (1, 2, 0, 3))
    mask = jnp.transpose(mask_t.reshape(H, B, W), (1, 0, 2))
    loss = jnp.transpose(loss_t.reshape(H, B, W), (1, 0, 2))
    return (x_out, mask, loss)


# speedup vs baseline: 1.8981x; 1.0305x over previous
import jax
import jax.numpy as jnp
from jax.experimental import pallas as pl
from jax.experimental.pallas import tpu as pltpu

C = 128
H = 112
W = 152
INV_C = 1.0 / C
_NCH = 8
_CC = C // _NCH
_RR = _CC * 32


def _leaky(v):
    return jnp.where(v >= 0, v, 0.01 * v)


def _row_kernel(x_ref, xg_ref, wc_ref, bc_ref, w1_ref, w2_ref, wrc_ref,
                xo_ref, mask_ref, loss_ref):
    T = x_ref.shape[2]
    Bn = xo_ref.shape[1]
    for r in range(x_ref.shape[0]):
        _one_row(r, x_ref, xg_ref, wc_ref, bc_ref, w1_ref, w2_ref, wrc_ref,
                 xo_ref, mask_ref, loss_ref, T, Bn)


def _one_row(r, x_ref, xg_ref, wc_ref, bc_ref, w1_ref, w2_ref, wrc_ref,
             xo_ref, mask_ref, loss_ref, T, Bn):
    X = x_ref[r]
    xg = xg_ref[r]
    idx = jnp.clip((xg * C).astype(jnp.int32), 0, C - 1)
    ci = jax.lax.broadcasted_iota(jnp.int32, (C, T), 0)
    oh = (ci == idx).astype(jnp.float32)

    cls = jnp.dot(wc_ref[r], X, preferred_element_type=jnp.float32)
    cls = _leaky(cls + bc_ref[r])
    mx = jnp.max(cls, axis=0, keepdims=True)
    e = jnp.exp(cls - mx)
    p = e / jnp.sum(e, axis=0, keepdims=True)
    lse = jnp.log(jnp.sum(jnp.exp(p), axis=0, keepdims=True))
    p_gt = jnp.sum(p * oh, axis=0, keepdims=True)
    loss_ref[r] = lse - p_gt

    y1 = jnp.zeros((32, T), jnp.float32)
    for k in range(_NCH):
        mk = jnp.dot(w1_ref[r, k * _RR:(k + 1) * _RR, :], X,
                     preferred_element_type=jnp.float32)
        mk3 = mk.reshape(_CC, 32, T)
        ohk = oh[k * _CC:(k + 1) * _CC, :]
        y1 = y1 + jnp.sum(mk3 * ohk[:, None, :], axis=0)
    y1 = _leaky(y1)

    g2 = jax.lax.dot_general(w2_ref[r], oh, (((0,), (0,)), ((), ())),
                             preferred_element_type=jnp.float32)
    g2v = g2.reshape(32, 32, T)
    y2 = _leaky(jnp.sum(g2v * y1[None, :, :], axis=1))

    g3 = jax.lax.dot_general(wrc_ref[r], oh, (((0,), (0,)), ((), ())),
                             preferred_element_type=jnp.float32)
    g3v = g3.reshape(2, 32, T)
    y3 = jnp.sum(g3v * y2[None, :, :], axis=1)
    reg = _leaky(y3[0])
    mask_ref[r] = _leaky(y3[1])[None, :]
    idxf = idx[0].astype(jnp.float32)

    for i in range(Bn):
        for j in range(Bn):
            xo_ref[r, i, j, :] = (idxf[i * W:(i + 1) * W] * INV_C
                                  + reg[j * W:(j + 1) * W] * INV_C)


def kernel(x, x_gt, conv_c_w, conv_c_b, w1, w2, wrc):
    B = x.shape[0]
    T = B * W
    xr = jnp.transpose(x, (2, 1, 0, 3)).reshape(H, 128, T)
    xgr = jnp.transpose(x_gt, (2, 1, 0, 3)).reshape(H, 1, T)
    wc = conv_c_w.reshape(H, C, 128)
    bc = conv_c_b.reshape(H, C)[:, :, None]
    w1n = w1.reshape(H, C, 128, 32).transpose(0, 1, 3, 2).reshape(H, C * 32, 128)
    w2g = w2.reshape(H, C, 32, 32).transpose(0, 1, 3, 2).reshape(H, C, 32 * 32)
    wrcg = wrc.reshape(H, C, 32, 2).transpose(0, 1, 3, 2).reshape(H, C, 2 * 32)

    R = 4
    xo_t, mask_t, loss_t = pl.pallas_call(
        _row_kernel,
        grid=(H // R,),
        in_specs=[
            pl.BlockSpec((R, 128, T), lambda h: (h, 0, 0)),
            pl.BlockSpec((R, 1, T), lambda h: (h, 0, 0)),
            pl.BlockSpec((R, C, 128), lambda h: (h, 0, 0)),
            pl.BlockSpec((R, C, 1), lambda h: (h, 0, 0)),
            pl.BlockSpec((R, C * 32, 128), lambda h: (h, 0, 0)),
            pl.BlockSpec((R, C, 32 * 32), lambda h: (h, 0, 0)),
            pl.BlockSpec((R, C, 2 * 32), lambda h: (h, 0, 0)),
        ],
        out_specs=[
            pl.BlockSpec((R, B, B, W), lambda h: (h, 0, 0, 0)),
            pl.BlockSpec((R, 1, T), lambda h: (h, 0, 0)),
            pl.BlockSpec((R, 1, T), lambda h: (h, 0, 0)),
        ],
        out_shape=[
            jax.ShapeDtypeStruct((H, B, B, W), jnp.float32),
            jax.ShapeDtypeStruct((H, 1, T), jnp.float32),
            jax.ShapeDtypeStruct((H, 1, T), jnp.float32),
        ],
        compiler_params=pltpu.CompilerParams(
            dimension_semantics=("parallel",)),
    )(xr, xgr, wc, bc, w1n, w2g, wrcg)

    x_out = jnp.transpose(xo_t, (1, 2, 0, 3))
    mask = jnp.transpose(mask_t.reshape(H, B, W), (1, 0, 2))
    loss = jnp.transpose(loss_t.reshape(H, B, W), (1, 0, 2))
    return (x_out, mask, loss)
